# baseline (speedup 1.0000x reference)
import functools

import jax
import jax.numpy as jnp
from jax import lax
from jax.experimental import pallas as pl
from jax.experimental.pallas import tpu as pltpu
from jax.experimental.pallas import tpu_sc as plsc

B, C, H, W = 4, 384, 64, 64
N = 100
RS = 7
HW = H * W
HWB = B * HW
D_AUG = 512
NT = 8
TW = HW // NT
RT = H // NT
NWORK = 32
NPIX_TILE = HWB // NWORK
CHUNK = 64
NCHUNK = NPIX_TILE // CHUNK
CG = C // 16
SH, SW = 8, HW // 8


def _proj_body(fr_ref, ft_ref, wr_ref, wt_ref, or_ref, ot_ref):
    pad = jnp.zeros((TW, D_AUG - C - 2), jnp.float32)
    f_r = jnp.concatenate(
        [fr_ref[0, :, r, :].T for r in range(RT)], axis=0)
    f_t = jnp.concatenate(
        [ft_ref[0, :, r, :].T for r in range(RT)], axis=0)
    lg_r = jnp.dot(f_r, wr_ref[...].T, preferred_element_type=jnp.float32)
    lg_t = jnp.dot(f_t, wt_ref[...].T, preferred_element_type=jnp.float32)
    or_ref[0] = jnp.concatenate([f_r, lg_r, pad], axis=1)
    ot_ref[0] = jnp.concatenate([f_t, lg_t, pad], axis=1)


def _project(feat_rgb, feat_tir, w_r, w_t):
    out_shape = [
        jax.ShapeDtypeStruct((B, HW, D_AUG), jnp.float32),
        jax.ShapeDtypeStruct((B, HW, D_AUG), jnp.float32),
    ]
    aug_r, aug_t = pl.pallas_call(
        _proj_body,
        grid=(B, NT),
        in_specs=[
            pl.BlockSpec((1, C, RT, W), lambda b, t: (b, 0, t, 0)),
            pl.BlockSpec((1, C, RT, W), lambda b, t: (b, 0, t, 0)),
            pl.BlockSpec((2, C), lambda b, t: (0, 0)),
            pl.BlockSpec((2, C), lambda b, t: (0, 0)),
        ],
        out_specs=[
            pl.BlockSpec((1, TW, D_AUG), lambda b, t: (b, t, 0)),
            pl.BlockSpec((1, TW, D_AUG), lambda b, t: (b, t, 0)),
        ],
        out_shape=out_shape,
    )(feat_rgb, feat_tir, w_r, w_t)
    return aug_r.reshape(HWB, D_AUG), aug_t.reshape(HWB, D_AUG)


def _idx_body(ar_ref, at_ref, ir_ref, it_ref, mint_ref,
              iyr_ref, ixr_ref, iyt_ref, ixt_ref, cov_ref):
    b = pl.program_id(0)
    i0 = lax.broadcasted_iota(jnp.int32, (SH, SW), 0)
    i1 = lax.broadcasted_iota(jnp.int32, (SH, SW), 1)
    p = i0 * SW + i1
    row = p >> 6
    col = p & (W - 1)
    iyr_ref[...] = row
    ixr_ref[...] = col
    iyt_ref[...] = row
    ixt_ref[...] = col
    cov_ref[...] = jnp.zeros((SH, SW), jnp.int32)

    def body(r, _):
        axr = ar_ref[0, r, 0]
        ayr = ar_ref[0, r, 1]
        axt = at_ref[0, r, 0]
        ayt = at_ref[0, r, 1]
        y0 = jnp.clip(ayr - 4, 0, H - RS)
        x0 = jnp.clip(axr - 4, 0, W - RS)
        cover = (row >= y0) & (row < y0 + RS) & (col >= x0) & (col < x0 + RS)
        iyr_ref[...] = jnp.where(cover, ayr - 3 + (row - y0), iyr_ref[...])
        ixr_ref[...] = jnp.where(cover, axr - 3 + (col - x0), ixr_ref[...])
        iyt_ref[...] = jnp.where(cover, ayt - 3 + (row - y0), iyt_ref[...])
        ixt_ref[...] = jnp.where(cover, axt - 3 + (col - x0), ixt_ref[...])
        cov_ref[...] = jnp.where(cover, 1, cov_ref[...])
        return 0

    lax.fori_loop(0, N, body, 0)
    iyr = iyr_ref[...]
    ixr = ixr_ref[...]
    iyt = iyt_ref[...]
    ixt = ixt_ref[...]
    sel = (cov_ref[...] > 0).astype(jnp.int32)
    mr = ((iyr >= -1) & (ixr >= -1)).astype(jnp.int32)
    mt = ((iyt >= -1) & (ixt >= -1)).astype(jnp.int32)
    base = b * HW
    ir_ref[0] = base + jnp.maximum(iyr, 0) * W + jnp.maximum(ixr, 0)
    it_ref[0] = base + jnp.maximum(iyt, 0) * W + jnp.maximum(ixt, 0)
    mint_ref[0] = mr | (mt << 1) | (sel << 2)


def _indices(anc_rgb, anc_tir):
    out_shape = [
        jax.ShapeDtypeStruct((B, SH, SW), jnp.int32),
        jax.ShapeDtypeStruct((B, SH, SW), jnp.int32),
        jax.ShapeDtypeStruct((B, SH, SW), jnp.int32),
    ]
    outs = pl.pallas_call(
        _idx_body,
        grid=(B,),
        in_specs=[
            pl.BlockSpec((1, N, 2), lambda b: (b, 0, 0),
                         memory_space=pltpu.SMEM),
            pl.BlockSpec((1, N, 2), lambda b: (b, 0, 0),
                         memory_space=pltpu.SMEM),
        ],
        out_specs=[pl.BlockSpec((1, SH, SW), lambda b: (b, 0, 0))] * 3,
        out_shape=out_shape,
        scratch_shapes=[pltpu.VMEM((SH, SW), jnp.int32)] * 5,
    )(anc_rgb, anc_tir)
    return tuple(o.reshape(HWB) for o in outs)


def _sc_body(aug_r, aug_t, idx_r, idx_t, mint, bias2,
             gout_r, gout_t, gout_wr, gout_wt,
             idxr_v, idxt_v, mint_v, bias_v, rows_r, rows_t,
             wr_v, wt_v, sem_r, sem_t):
    wid = lax.axis_index("s") * 2 + lax.axis_index("c")
    tbase = wid * NPIX_TILE
    pltpu.sync_copy(bias2, bias_v)

    def chunk_body(ci, _):
        base = tbase + ci * CHUNK
        pltpu.sync_copy(idx_r.at[pl.ds(base, CHUNK)], idxr_v)
        pltpu.sync_copy(idx_t.at[pl.ds(base, CHUNK)], idxt_v)
        pltpu.sync_copy(mint.at[pl.ds(base, CHUNK)], mint_v)
        cp_r = pltpu.async_copy(aug_r.at[idxr_v], rows_r, sem_r)
        cp_t = pltpu.async_copy(aug_t.at[idxt_v], rows_t, sem_t)
        cp_r.wait()
        cp_t.wait()

        def pix_body(p, _):
            lg_r = rows_r[p, pl.ds(C, 16)]
            lg_t = rows_t[p, pl.ds(C, 16)]
            m = mint_v[pl.ds(p, 1)][0]
            bv = bias_v[...]
            mr = (m & 1).astype(jnp.float32)
            mt = ((m >> 1) & 1).astype(jnp.float32)
            s = (m >> 2) & 1
            d_roi = mr * lg_r[0] + mt * lg_t[0]
            d_glb = lg_r[1] + lg_t[1]
            bd = jnp.where(s == 1, bv[0], bv[1])
            d = jnp.where(s == 1, d_roi, d_glb) + bd
            dv = lax.broadcast(d, (16,))
            alpha = 1.0 / (1.0 + jnp.exp(-dv))
            wr = lax.broadcast(mr, (16,)) * alpha
            wt = lax.broadcast(mt, (16,)) * (1.0 - alpha)
            wr_v[pl.ds(p * 16, 16)] = wr
            wt_v[pl.ds(p * 16, 16)] = wt
            return 0

        lax.fori_loop(0, CHUNK, pix_body, 0)
        pltpu.sync_copy(rows_r, gout_r.at[pl.ds(base, CHUNK)])
        pltpu.sync_copy(rows_t, gout_t.at[pl.ds(base, CHUNK)])
        pltpu.sync_copy(wr_v, gout_wr.at[pl.ds(base * 16, CHUNK * 16)])
        pltpu.sync_copy(wt_v, gout_wt.at[pl.ds(base * 16, CHUNK * 16)])
        return 0

    lax.fori_loop(0, NCHUNK, chunk_body, 0)


def _sc_blend(aug_r, aug_t, idx_r, idx_t, mint, bias2):
    mesh = plsc.VectorSubcoreMesh(core_axis_name="c", subcore_axis_name="s")
    run = functools.partial(
        pl.kernel,
        mesh=mesh,
        out_type=[
            jax.ShapeDtypeStruct((HWB, D_AUG), jnp.float32),
            jax.ShapeDtypeStruct((HWB, D_AUG), jnp.float32),
            jax.ShapeDtypeStruct((HWB * 16,), jnp.float32),
            jax.ShapeDtypeStruct((HWB * 16,), jnp.float32),
        ],
        scratch_types=[
            pltpu.VMEM((CHUNK,), jnp.int32),
            pltpu.VMEM((CHUNK,), jnp.int32),
            pltpu.VMEM((CHUNK,), jnp.int32),
            pltpu.VMEM((16,), jnp.float32),
            pltpu.VMEM((CHUNK, D_AUG), jnp.float32),
            pltpu.VMEM((CHUNK, D_AUG), jnp.float32),
            pltpu.VMEM((CHUNK * 16,), jnp.float32),
            pltpu.VMEM((CHUNK * 16,), jnp.float32),
            pltpu.SemaphoreType.DMA,
            pltpu.SemaphoreType.DMA,
        ],
    )(_sc_body)
    return run(aug_r, aug_t, idx_r, idx_t, mint, bias2)


def _bt_body(gr_ref, gt_ref, wr_ref, wt_ref, o_ref):
    gr = gr_ref[0]
    gt = gt_ref[0]
    wr = wr_ref[0][:, 0:1]
    wt = wt_ref[0][:, 0:1]
    out = gr[:, :C] * wr + gt[:, :C] * wt
    for r in range(RT):
        o_ref[0, :, r, :] = out[r * W:(r + 1) * W, :].T


def _blend_tr(gr, gt, wrb, wtb):
    return pl.pallas_call(
        _bt_body,
        grid=(B, NT),
        in_specs=[
            pl.BlockSpec((1, TW, D_AUG), lambda b, t: (b, t, 0)),
            pl.BlockSpec((1, TW, D_AUG), lambda b, t: (b, t, 0)),
            pl.BlockSpec((1, TW, 16), lambda b, t: (b, t, 0)),
            pl.BlockSpec((1, TW, 16), lambda b, t: (b, t, 0)),
        ],
        out_specs=pl.BlockSpec((1, C, RT, W), lambda b, t: (b, 0, t, 0)),
        out_shape=jax.ShapeDtypeStruct((B, C, H, W), jnp.float32),
    )(gr.reshape(B, HW, D_AUG), gt.reshape(B, HW, D_AUG),
      wrb.reshape(B, HW, 16), wtb.reshape(B, HW, 16))


def kernel(feat_rgb, feat_tir, anchors_rgb_with_conf, anchors_tir_with_conf,
           Wg, bg, Wa, ba):
    anc_rgb = anchors_rgb_with_conf[..., :2].astype(jnp.int32)
    anc_tir = anchors_tir_with_conf[..., :2].astype(jnp.int32)
    w_r = jnp.stack([Wa[0, :C] - Wa[1, :C], Wg[0, :C] - Wg[1, :C]], axis=0)
    w_t = jnp.stack([Wa[0, C:] - Wa[1, C:], Wg[0, C:] - Wg[1, C:]], axis=0)
    bias2 = jnp.pad(jnp.stack([ba[0] - ba[1], bg[0] - bg[1]]), (0, 14))

    aug_r, aug_t = _project(feat_rgb, feat_tir, w_r, w_t)
    idx_r, idx_t, mint = _indices(anc_rgb, anc_tir)
    gath_r, gath_t, wrb, wtb = _sc_blend(aug_r, aug_t, idx_r, idx_t, mint,
                                         bias2)
    return _blend_tr(gath_r, gath_t, wrb, wtb)
